# baseline (speedup 1.0000x reference)
import functools

import jax
import jax.numpy as jnp
from jax import lax
from jax.experimental import pallas as pl
from jax.experimental.pallas import tpu as pltpu
from jax.experimental.pallas import tpu_sc as plsc

N_NODES = 10000
N_EDGES = 160000
DIM = 256
HALF = 128
N_GRAPHS = 64

EB = 64
NBLK = N_EDGES // EB
NSUB = 16
BLK_T = NBLK // NSUB
EDGE_T = BLK_T * EB
NREST = NBLK - BLK_T * NSUB
NFULL = N_NODES // EB
NTAIL = N_NODES - NFULL * EB

def _edge_sc_body(h_lo, h_hi, e_lo, e_hi, src, dst3,
                  out_lo, out_hi, ibs, ibd, stail, dtail,
                  gbuf2, ebuf2, acc,
                  sg0, sg1, sg2, se0, se1, se2,
                  ss0, ss1, ss2,
                  sis0, sis1, sis2, sis3, sid0, sid1, sid2, sid3):
  c = lax.axis_index("c")
  s = lax.axis_index("s")
  gbuf = gbuf2.at[0]

  def zero_row(r, _):
    for k in range(8):
      gbuf[r, pl.ds(k * 16, 16)] = jnp.zeros((16,), jnp.float32)
    return 0
  lax.fori_loop(0, EB, zero_row, 0)
  for t in range(10):
    m = s + NSUB * t

    @pl.when(m < NFULL)
    def _():
      pltpu.sync_copy(gbuf, acc.at[pl.ds(m * EB, EB)])

  @pl.when(s == NFULL % NSUB)
  def _():
    pltpu.sync_copy(gbuf.at[pl.ds(0, NTAIL)], acc.at[pl.ds(NFULL * EB, NTAIL)])
  plsc.subcore_barrier()

  base = s * EDGE_T

  sg = (sg0, sg1, sg2)
  se = (se0, se1, se2)
  ss = (ss0, ss1, ss2)
  sis = (sis0, sis1, sis2, sis3)
  sid = (sid0, sid1, sid2, sid3)
  himask = jnp.full((16,), -65536, jnp.int32)
  shift = jnp.full((16,), 16, jnp.int32)

  def compute(buf):
    @plsc.parallel_loop(0, EB, 1, unroll=2)
    def _(r):
      for g in range(4):
        ei = ebuf2[buf, r, pl.ds(16 * g, 16)]
        ea = lax.bitcast_convert_type(lax.shift_left(ei, shift), jnp.float32)
        ebv = lax.bitcast_convert_type(lax.bitwise_and(ei, himask),
                                       jnp.float32)
        sla = pl.ds(32 * g, 16)
        slb = pl.ds(32 * g + 16, 16)
        gbuf2[buf, r, sla] = jnp.maximum(gbuf2[buf, r, sla] + ea, 0.0)
        gbuf2[buf, r, slb] = jnp.maximum(gbuf2[buf, r, slb] + ebv, 0.0)

  def run_core(h_tab, e_tab, out_ref):
    def sidx_desc(j, ib):
      return pltpu.make_async_copy(
          src.at[pl.ds(pl.multiple_of(base + j * EB, 32), EB)],
          ibs.at[ib], sis[ib])

    def didx_desc(j, ib):
      return pltpu.make_async_copy(
          dst3.at[pl.ds(s * BLK_T + j, 1)], ibd.at[ib], sid[ib])

    def ge_descs(j, ib, buf):
      return (
          pltpu.make_async_copy(
              h_tab.at[ibs.at[ib]], gbuf2.at[buf], sg[buf]),
          pltpu.make_async_copy(
              e_tab.at[pl.ds(pl.multiple_of(base + j * EB, 32), EB)],
              ebuf2.at[buf], se[buf]),
      )

    def sub_body(j, u):
      b = u % 3
      nb = (u + 2) % 3
      i4 = u % 4
      n4 = (u + 2) % 4
      p4 = (u + 3) % 4
      for d in ge_descs(j, i4, b):
        d.wait()
      compute(b)
      if u == 0:
        @pl.when(j > 0)
        def _():
          pltpu.make_async_copy(
              gbuf2.at[nb], acc.at[ibd.at[p4, 0, 0]], ss[nb]).wait()
      else:
        pltpu.make_async_copy(
            gbuf2.at[nb], acc.at[ibd.at[p4, 0, 0]], ss[nb]).wait()
      @pl.when(j + 2 < BLK_T)
      def _():
        sidx_desc(j + 2, n4).wait()
        for d in ge_descs(j + 2, n4, nb):
          d.start()
        didx_desc(j + 2, n4).start()

      @pl.when(j + 4 < BLK_T)
      def _():
        sidx_desc(j + 4, i4).start()
      didx_desc(j, i4).wait()
      pltpu.async_copy(gbuf2.at[b], acc.at[ibd.at[i4, 0, 0]], ss[b], add=True)

    sidx_desc(0, 0).start()
    sidx_desc(1, 1).start()
    didx_desc(0, 0).start()
    didx_desc(1, 1).start()
    sidx_desc(0, 0).wait()
    for d in ge_descs(0, 0, 0):
      d.start()
    sidx_desc(2, 2).start()
    sidx_desc(1, 1).wait()
    for d in ge_descs(1, 1, 1):
      d.start()
    sidx_desc(3, 3).start()

    def body(i, _):
      for u in range(12):
        sub_body(12 * i + u, u)
      return 0
    lax.fori_loop(0, BLK_T // 12, body, 0)
    pltpu.make_async_copy(
        gbuf2.at[2], acc.at[ibd.at[3, 0, 0]], ss[2]).wait()

    @pl.when(s < NREST)
    def _():
      g = BLK_T * NSUB + s
      off = g * EB
      pltpu.sync_copy(src.at[pl.ds(off, EB)], stail)
      pltpu.sync_copy(dst3.at[pl.ds(g, 1)], dtail)
      pltpu.async_copy(h_tab.at[stail], gbuf2.at[0], sg0).wait()
      pltpu.sync_copy(e_tab.at[pl.ds(off, EB)], ebuf2.at[0])
      compute(0)
      pltpu.sync_copy(gbuf2.at[0], acc.at[dtail.at[0, 0]], add=True)

  @pl.when(c == 0)
  def _():
    run_core(h_lo, e_lo, out_lo)

  @pl.when(c == 1)
  def _():
    run_core(h_hi, e_hi, out_hi)

  plsc.subcore_barrier()

  for t in range(10):
    m = s + NSUB * t

    @pl.when(m < NFULL)
    def _():
      r0 = m * EB
      pltpu.sync_copy(acc.at[pl.ds(r0, EB)], gbuf)

      @pl.when(c == 0)
      def _():
        pltpu.sync_copy(gbuf, out_lo.at[pl.ds(r0, EB)])

      @pl.when(c == 1)
      def _():
        pltpu.sync_copy(gbuf, out_hi.at[pl.ds(r0, EB)])

  @pl.when(s == NFULL % NSUB)
  def _():
    r0 = NFULL * EB
    pltpu.sync_copy(acc.at[pl.ds(r0, NTAIL)], gbuf.at[pl.ds(0, NTAIL)])

    @pl.when(c == 0)
    def _():
      pltpu.sync_copy(gbuf.at[pl.ds(0, NTAIL)], out_lo.at[pl.ds(r0, NTAIL)])

    @pl.when(c == 1)
    def _():
      pltpu.sync_copy(gbuf.at[pl.ds(0, NTAIL)], out_hi.at[pl.ds(r0, NTAIL)])


_edge_sc = pl.kernel(
    _edge_sc_body,
    out_type=(jax.ShapeDtypeStruct((N_NODES, HALF), jnp.float32),
              jax.ShapeDtypeStruct((N_NODES, HALF), jnp.float32)),
    mesh=plsc.VectorSubcoreMesh(core_axis_name="c", subcore_axis_name="s"),
    scratch_types=[
        pltpu.VMEM((4, EB), jnp.int32),
        pltpu.VMEM((4, 1, 1, EB), jnp.int32),
        pltpu.VMEM((EB,), jnp.int32),
        pltpu.VMEM((1, 1, EB), jnp.int32),
        pltpu.VMEM((3, EB, HALF), jnp.float32),
        pltpu.VMEM((3, EB, HALF // 2), jnp.int32),
        pltpu.VMEM_SHARED((N_NODES, HALF), jnp.float32),
    ] + [pltpu.SemaphoreType.DMA] * 17,
)


def _pack_bf16_pairs(eh):
  a = jnp.concatenate([eh[:, 32 * g:32 * g + 16] for g in range(4)], axis=1)
  b = jnp.concatenate([eh[:, 32 * g + 16:32 * g + 32] for g in range(4)],
                      axis=1)
  ai = lax.bitcast_convert_type(
      a.astype(jnp.bfloat16).astype(jnp.float32), jnp.int32)
  bi = lax.bitcast_convert_type(
      b.astype(jnp.bfloat16).astype(jnp.float32), jnp.int32)
  return lax.bitwise_or(lax.bitwise_and(bi, jnp.int32(-65536)),
                        lax.shift_right_logical(ai, 16))


def _edge_lin_body(ea_ref, we_ref, be_ref, olo_ref, ohi_ref):
  e = lax.dot_general(ea_ref[...], we_ref[...], (((1,), (1,)), ((), ())),
                      preferred_element_type=jnp.float32) + be_ref[...]
  olo_ref[...] = _pack_bf16_pairs(e[:, :HALF])
  ohi_ref[...] = _pack_bf16_pairs(e[:, HALF:])


def _edge_lin(ea, we, be):
  BE = 2000
  grid = (N_EDGES // BE,)
  return pl.pallas_call(
      _edge_lin_body,
      grid=grid,
      in_specs=[
          pl.BlockSpec((BE, 16), lambda i: (i, 0)),
          pl.BlockSpec((DIM, 16), lambda i: (0, 0)),
          pl.BlockSpec((1, DIM), lambda i: (0, 0)),
      ],
      out_specs=[
          pl.BlockSpec((BE, HALF // 2), lambda i: (i, 0)),
          pl.BlockSpec((BE, HALF // 2), lambda i: (i, 0)),
      ],
      out_shape=[jax.ShapeDtypeStruct((N_EDGES, HALF // 2), jnp.int32),
                 jax.ShapeDtypeStruct((N_EDGES, HALF // 2), jnp.int32)],
  )(ea, we, be.reshape(1, DIM))


def _mlp_body(hlo_ref, hhi_ref, alo_ref, ahi_ref, w1_ref, b1_ref,
              w2_ref, b2_ref, olo_ref, ohi_ref):
  z = jnp.concatenate([hlo_ref[...] + alo_ref[...],
                       hhi_ref[...] + ahi_ref[...]], axis=1)
  y = lax.dot_general(z, w1_ref[...], (((1,), (1,)), ((), ())),
                      preferred_element_type=jnp.float32) + b1_ref[...]
  y = jnp.maximum(y, 0.0)
  y = lax.dot_general(y, w2_ref[...], (((1,), (1,)), ((), ())),
                      preferred_element_type=jnp.float32) + b2_ref[...]
  h = jnp.maximum(y, 0.0)
  olo_ref[...] = h[:, :HALF]
  ohi_ref[...] = h[:, HALF:]


def _mlp(hlo, hhi, alo, ahi, w1, b1, w2, b2):
  BN = 2000
  grid = (N_NODES // BN,)
  half_spec = pl.BlockSpec((BN, HALF), lambda i: (i, 0))
  w_spec = pl.BlockSpec((DIM, DIM), lambda i: (0, 0))
  b_spec = pl.BlockSpec((1, DIM), lambda i: (0, 0))
  return pl.pallas_call(
      _mlp_body,
      grid=grid,
      in_specs=[half_spec, half_spec, half_spec, half_spec,
                w_spec, b_spec, w_spec, b_spec],
      out_specs=[half_spec, half_spec],
      out_shape=[jax.ShapeDtypeStruct((N_NODES, HALF), jnp.float32),
                 jax.ShapeDtypeStruct((N_NODES, HALF), jnp.float32)],
  )(hlo, hhi, alo, ahi, w1, b1.reshape(1, DIM), w2, b2.reshape(1, DIM))


def _pool_body(hlo_ref, hhi_ref, b_ref, wf1_ref, bf1_ref, wf2_ref, bf2_ref,
               o_ref):
  h = jnp.concatenate([hlo_ref[...], hhi_ref[...]], axis=1)
  gids = lax.broadcasted_iota(jnp.int32, (1, N_GRAPHS), 1)
  onehot = (b_ref[...] == gids).astype(jnp.float32)
  pooled = lax.dot_general(onehot, h, (((0,), (0,)), ((), ())),
                           preferred_element_type=jnp.float32)
  p = lax.dot_general(pooled, wf1_ref[...], (((1,), (1,)), ((), ())),
                      preferred_element_type=jnp.float32) + bf1_ref[...]
  p = jnp.maximum(p, 0.0)
  o = lax.dot_general(p, wf2_ref[...], (((1,), (1,)), ((), ())),
                      preferred_element_type=jnp.float32) + bf2_ref[0, 0]
  o_ref[...] = o


def _pool(hlo, hhi, batch, wf1, bf1, wf2, bf2):
  wf2p = jnp.zeros((8, DIM), jnp.float32).at[0].set(wf2[0])
  return pl.pallas_call(
      _pool_body,
      out_shape=jax.ShapeDtypeStruct((N_GRAPHS, 8), jnp.float32),
  )(hlo, hhi, batch.reshape(N_NODES, 1), wf1, bf1.reshape(1, DIM),
    wf2p, bf2.reshape(1, 1))


def kernel(x, edge_index, edge_attr, batch,
           We0, be0, W10, b10, W20, b20,
           We1, be1, W11, b11, W21, b21,
           We2, be2, W12, b12, W22, b22,
           Wf1, bf1, Wf2, bf2):
  src = edge_index[0]
  dst3 = edge_index[1].reshape(NBLK, 1, EB)
  layers = [(We0, be0, W10, b10, W20, b20),
            (We1, be1, W11, b11, W21, b21),
            (We2, be2, W12, b12, W22, b22)]
  hlo = x[:, :HALF]
  hhi = x[:, HALF:]
  eterms = [_edge_lin(edge_attr, We, be) for (We, be, _, _, _, _) in layers]
  for (e_lo, e_hi), (We, be, W1, b1, W2, b2) in zip(eterms, layers):
    a_lo, a_hi = _edge_sc(hlo, hhi, e_lo, e_hi, src, dst3)
    hlo, hhi = _mlp(hlo, hhi, a_lo, a_hi, W1, b1, W2, b2)
  out = _pool(hlo, hhi, batch, Wf1, bf1, Wf2, bf2)
  return out[:, 0]


# speedup vs baseline: 1.0056x; 1.0056x over previous
import functools

import jax
import jax.numpy as jnp
from jax import lax
from jax.experimental import pallas as pl
from jax.experimental.pallas import tpu as pltpu
from jax.experimental.pallas import tpu_sc as plsc

N_NODES = 10000
N_EDGES = 160000
DIM = 256
HALF = 128
N_GRAPHS = 64

EB = 64
NBLK = N_EDGES // EB
NSUB = 16
BLK_T = NBLK // NSUB
EDGE_T = BLK_T * EB
NREST = NBLK - BLK_T * NSUB
NFULL = N_NODES // EB
NTAIL = N_NODES - NFULL * EB

def _edge_sc_body(h_lo, h_hi, e_lo, e_hi, src, dst3,
                  out_lo, out_hi, ibs, ibd, stail, dtail,
                  gbuf2, ebuf2, acc,
                  sg0, sg1, sg2, se0, se1, se2,
                  ss0, ss1, ss2,
                  sis0, sis1, sis2, sis3, sid0, sid1, sid2, sid3):
  c = lax.axis_index("c")
  s = lax.axis_index("s")
  gbuf = gbuf2.at[0]

  def zero_row(r, _):
    for k in range(8):
      gbuf[r, pl.ds(k * 16, 16)] = jnp.zeros((16,), jnp.float32)
    return 0
  lax.fori_loop(0, EB, zero_row, 0)
  for t in range(10):
    m = s + NSUB * t

    @pl.when(m < NFULL)
    def _():
      pltpu.sync_copy(gbuf, acc.at[pl.ds(m * EB, EB)])

  @pl.when(s == NFULL % NSUB)
  def _():
    pltpu.sync_copy(gbuf.at[pl.ds(0, NTAIL)], acc.at[pl.ds(NFULL * EB, NTAIL)])
  plsc.subcore_barrier()

  base = s * EDGE_T

  sg = (sg0, sg1, sg2)
  se = (se0, se1, se2)
  ss = (ss0, ss1, ss2)
  sis = (sis0, sis1, sis2, sis3)
  sid = (sid0, sid1, sid2, sid3)
  himask = jnp.full((16,), -65536, jnp.int32)
  shift = jnp.full((16,), 16, jnp.int32)

  def compute(buf):
    def rows(r2, _):
      for dr in range(2):
        r = r2 * 2 + dr
        for g in range(4):
          ei = ebuf2[buf, r, pl.ds(16 * g, 16)]
          ea = lax.bitcast_convert_type(lax.shift_left(ei, shift), jnp.float32)
          ebv = lax.bitcast_convert_type(lax.bitwise_and(ei, himask),
                                         jnp.float32)
          sla = pl.ds(32 * g, 16)
          slb = pl.ds(32 * g + 16, 16)
          gbuf2[buf, r, sla] = jnp.maximum(gbuf2[buf, r, sla] + ea, 0.0)
          gbuf2[buf, r, slb] = jnp.maximum(gbuf2[buf, r, slb] + ebv, 0.0)
      return 0
    lax.fori_loop(0, EB // 2, rows, 0)

  def run_core(h_tab, e_tab, out_ref):
    def sidx_desc(j, ib):
      return pltpu.make_async_copy(
          src.at[pl.ds(pl.multiple_of(base + j * EB, 32), EB)],
          ibs.at[ib], sis[ib])

    def didx_desc(j, ib):
      return pltpu.make_async_copy(
          dst3.at[pl.ds(s * BLK_T + j, 1)], ibd.at[ib], sid[ib])

    def ge_descs(j, ib, buf):
      return (
          pltpu.make_async_copy(
              h_tab.at[ibs.at[ib]], gbuf2.at[buf], sg[buf]),
          pltpu.make_async_copy(
              e_tab.at[pl.ds(pl.multiple_of(base + j * EB, 32), EB)],
              ebuf2.at[buf], se[buf]),
      )

    def sub_body(j, u):
      b = u % 3
      nb = (u + 2) % 3
      i4 = u % 4
      n4 = (u + 2) % 4
      p4 = (u + 3) % 4
      for d in ge_descs(j, i4, b):
        d.wait()
      compute(b)
      if u == 0:
        @pl.when(j > 0)
        def _():
          pltpu.make_async_copy(
              gbuf2.at[nb], acc.at[ibd.at[p4, 0, 0]], ss[nb]).wait()
      else:
        pltpu.make_async_copy(
            gbuf2.at[nb], acc.at[ibd.at[p4, 0, 0]], ss[nb]).wait()
      @pl.when(j + 2 < BLK_T)
      def _():
        sidx_desc(j + 2, n4).wait()
        for d in ge_descs(j + 2, n4, nb):
          d.start()
        didx_desc(j + 2, n4).start()

      @pl.when(j + 4 < BLK_T)
      def _():
        sidx_desc(j + 4, i4).start()
      didx_desc(j, i4).wait()
      pltpu.async_copy(gbuf2.at[b], acc.at[ibd.at[i4, 0, 0]], ss[b], add=True)

    sidx_desc(0, 0).start()
    sidx_desc(1, 1).start()
    didx_desc(0, 0).start()
    didx_desc(1, 1).start()
    sidx_desc(0, 0).wait()
    for d in ge_descs(0, 0, 0):
      d.start()
    sidx_desc(2, 2).start()
    sidx_desc(1, 1).wait()
    for d in ge_descs(1, 1, 1):
      d.start()
    sidx_desc(3, 3).start()

    def body(i, _):
      for u in range(12):
        sub_body(12 * i + u, u)
      return 0
    lax.fori_loop(0, BLK_T // 12, body, 0)
    pltpu.make_async_copy(
        gbuf2.at[2], acc.at[ibd.at[3, 0, 0]], ss[2]).wait()

    @pl.when(s < NREST)
    def _():
      g = BLK_T * NSUB + s
      off = g * EB
      pltpu.sync_copy(src.at[pl.ds(off, EB)], stail)
      pltpu.sync_copy(dst3.at[pl.ds(g, 1)], dtail)
      pltpu.async_copy(h_tab.at[stail], gbuf2.at[0], sg0).wait()
      pltpu.sync_copy(e_tab.at[pl.ds(off, EB)], ebuf2.at[0])
      compute(0)
      pltpu.sync_copy(gbuf2.at[0], acc.at[dtail.at[0, 0]], add=True)

  @pl.when(c == 0)
  def _():
    run_core(h_lo, e_lo, out_lo)

  @pl.when(c == 1)
  def _():
    run_core(h_hi, e_hi, out_hi)

  plsc.subcore_barrier()

  for t in range(10):
    m = s + NSUB * t

    @pl.when(m < NFULL)
    def _():
      r0 = m * EB
      pltpu.sync_copy(acc.at[pl.ds(r0, EB)], gbuf)

      @pl.when(c == 0)
      def _():
        pltpu.sync_copy(gbuf, out_lo.at[pl.ds(r0, EB)])

      @pl.when(c == 1)
      def _():
        pltpu.sync_copy(gbuf, out_hi.at[pl.ds(r0, EB)])

  @pl.when(s == NFULL % NSUB)
  def _():
    r0 = NFULL * EB
    pltpu.sync_copy(acc.at[pl.ds(r0, NTAIL)], gbuf.at[pl.ds(0, NTAIL)])

    @pl.when(c == 0)
    def _():
      pltpu.sync_copy(gbuf.at[pl.ds(0, NTAIL)], out_lo.at[pl.ds(r0, NTAIL)])

    @pl.when(c == 1)
    def _():
      pltpu.sync_copy(gbuf.at[pl.ds(0, NTAIL)], out_hi.at[pl.ds(r0, NTAIL)])


_edge_sc = pl.kernel(
    _edge_sc_body,
    out_type=(jax.ShapeDtypeStruct((N_NODES, HALF), jnp.float32),
              jax.ShapeDtypeStruct((N_NODES, HALF), jnp.float32)),
    mesh=plsc.VectorSubcoreMesh(core_axis_name="c", subcore_axis_name="s"),
    scratch_types=[
        pltpu.VMEM((4, EB), jnp.int32),
        pltpu.VMEM((4, 1, 1, EB), jnp.int32),
        pltpu.VMEM((EB,), jnp.int32),
        pltpu.VMEM((1, 1, EB), jnp.int32),
        pltpu.VMEM((3, EB, HALF), jnp.float32),
        pltpu.VMEM((3, EB, HALF // 2), jnp.int32),
        pltpu.VMEM_SHARED((N_NODES, HALF), jnp.float32),
    ] + [pltpu.SemaphoreType.DMA] * 17,
)


def _pack_bf16_pairs(eh):
  a = jnp.concatenate([eh[:, 32 * g:32 * g + 16] for g in range(4)], axis=1)
  b = jnp.concatenate([eh[:, 32 * g + 16:32 * g + 32] for g in range(4)],
                      axis=1)
  ai = lax.bitcast_convert_type(
      a.astype(jnp.bfloat16).astype(jnp.float32), jnp.int32)
  bi = lax.bitcast_convert_type(
      b.astype(jnp.bfloat16).astype(jnp.float32), jnp.int32)
  return lax.bitwise_or(lax.bitwise_and(bi, jnp.int32(-65536)),
                        lax.shift_right_logical(ai, 16))


def _edge_lin_body(ea_ref, we_ref, be_ref, olo_ref, ohi_ref):
  e = lax.dot_general(ea_ref[...], we_ref[...], (((1,), (1,)), ((), ())),
                      preferred_element_type=jnp.float32) + be_ref[...]
  olo_ref[...] = _pack_bf16_pairs(e[:, :HALF])
  ohi_ref[...] = _pack_bf16_pairs(e[:, HALF:])


def _edge_lin(ea, we, be):
  BE = 2000
  grid = (N_EDGES // BE,)
  return pl.pallas_call(
      _edge_lin_body,
      grid=grid,
      in_specs=[
          pl.BlockSpec((BE, 16), lambda i: (i, 0)),
          pl.BlockSpec((DIM, 16), lambda i: (0, 0)),
          pl.BlockSpec((1, DIM), lambda i: (0, 0)),
      ],
      out_specs=[
          pl.BlockSpec((BE, HALF // 2), lambda i: (i, 0)),
          pl.BlockSpec((BE, HALF // 2), lambda i: (i, 0)),
      ],
      out_shape=[jax.ShapeDtypeStruct((N_EDGES, HALF // 2), jnp.int32),
                 jax.ShapeDtypeStruct((N_EDGES, HALF // 2), jnp.int32)],
  )(ea, we, be.reshape(1, DIM))


def _mlp_body(hlo_ref, hhi_ref, alo_ref, ahi_ref, w1_ref, b1_ref,
              w2_ref, b2_ref, olo_ref, ohi_ref):
  z = jnp.concatenate([hlo_ref[...] + alo_ref[...],
                       hhi_ref[...] + ahi_ref[...]], axis=1)
  y = lax.dot_general(z, w1_ref[...], (((1,), (1,)), ((), ())),
                      preferred_element_type=jnp.float32) + b1_ref[...]
  y = jnp.maximum(y, 0.0)
  y = lax.dot_general(y, w2_ref[...], (((1,), (1,)), ((), ())),
                      preferred_element_type=jnp.float32) + b2_ref[...]
  h = jnp.maximum(y, 0.0)
  olo_ref[...] = h[:, :HALF]
  ohi_ref[...] = h[:, HALF:]


def _mlp(hlo, hhi, alo, ahi, w1, b1, w2, b2):
  BN = 2000
  grid = (N_NODES // BN,)
  half_spec = pl.BlockSpec((BN, HALF), lambda i: (i, 0))
  w_spec = pl.BlockSpec((DIM, DIM), lambda i: (0, 0))
  b_spec = pl.BlockSpec((1, DIM), lambda i: (0, 0))
  return pl.pallas_call(
      _mlp_body,
      grid=grid,
      in_specs=[half_spec, half_spec, half_spec, half_spec,
                w_spec, b_spec, w_spec, b_spec],
      out_specs=[half_spec, half_spec],
      out_shape=[jax.ShapeDtypeStruct((N_NODES, HALF), jnp.float32),
                 jax.ShapeDtypeStruct((N_NODES, HALF), jnp.float32)],
  )(hlo, hhi, alo, ahi, w1, b1.reshape(1, DIM), w2, b2.reshape(1, DIM))


def _pool_body(hlo_ref, hhi_ref, b_ref, wf1_ref, bf1_ref, wf2_ref, bf2_ref,
               o_ref):
  h = jnp.concatenate([hlo_ref[...], hhi_ref[...]], axis=1)
  gids = lax.broadcasted_iota(jnp.int32, (1, N_GRAPHS), 1)
  onehot = (b_ref[...] == gids).astype(jnp.float32)
  pooled = lax.dot_general(onehot, h, (((0,), (0,)), ((), ())),
                           preferred_element_type=jnp.float32)
  p = lax.dot_general(pooled, wf1_ref[...], (((1,), (1,)), ((), ())),
                      preferred_element_type=jnp.float32) + bf1_ref[...]
  p = jnp.maximum(p, 0.0)
  o = lax.dot_general(p, wf2_ref[...], (((1,), (1,)), ((), ())),
                      preferred_element_type=jnp.float32) + bf2_ref[0, 0]
  o_ref[...] = o


def _pool(hlo, hhi, batch, wf1, bf1, wf2, bf2):
  wf2p = jnp.zeros((8, DIM), jnp.float32).at[0].set(wf2[0])
  return pl.pallas_call(
      _pool_body,
      out_shape=jax.ShapeDtypeStruct((N_GRAPHS, 8), jnp.float32),
  )(hlo, hhi, batch.reshape(N_NODES, 1), wf1, bf1.reshape(1, DIM),
    wf2p, bf2.reshape(1, 1))


def kernel(x, edge_index, edge_attr, batch,
           We0, be0, W10, b10, W20, b20,
           We1, be1, W11, b11, W21, b21,
           We2, be2, W12, b12, W22, b22,
           Wf1, bf1, Wf2, bf2):
  src = edge_index[0]
  dst3 = edge_index[1].reshape(NBLK, 1, EB)
  layers = [(We0, be0, W10, b10, W20, b20),
            (We1, be1, W11, b11, W21, b21),
            (We2, be2, W12, b12, W22, b22)]
  hlo = x[:, :HALF]
  hhi = x[:, HALF:]
  eterms = [_edge_lin(edge_attr, We, be) for (We, be, _, _, _, _) in layers]
  for (e_lo, e_hi), (We, be, W1, b1, W2, b2) in zip(eterms, layers):
    a_lo, a_hi = _edge_sc(hlo, hhi, e_lo, e_hi, src, dst3)
    hlo, hhi = _mlp(hlo, hhi, a_lo, a_hi, W1, b1, W2, b2)
  out = _pool(hlo, hhi, batch, Wf1, bf1, Wf2, bf2)
  return out[:, 0]
